# baseline (speedup 1.0000x reference)
import functools

import jax
import jax.numpy as jnp
from jax import lax
from jax.experimental import pallas as pl
from jax.experimental.pallas import tpu as pltpu
from jax.experimental.pallas import tpu_sc as plsc

N = 10000
NP = 10240
E = 160000
NS = 16
K = 80
EPT = E // NS
DC = 128

_MESH = plsc.VectorSubcoreMesh(
    core_axis_name="c", subcore_axis_name="s", num_cores=1, num_subcores=NS)


def _zero_rows(buf, rows):
  def body(i, _):
    for w in range(DC // 16):
      buf[i, pl.ds(w * 16, 16)] = jnp.zeros((16,), jnp.float32)
    return 0
  lax.fori_loop(0, rows, body, 0)


NB = 5
_NSTEPS = EPT // K


def _make_sc(n_tables, n_ranges, rw):
  hd = rw + 128
  hrpt = hd // NS
  wrpt = rw // NS
  assert hd % 128 == 0 and wrpt % 8 == 0 and n_ranges * rw >= NP
  out_rows = n_ranges * rw
  n_out = max(n_tables, 1)
  ngrp = _NSTEPS // NB

  def body(src_hbm, dst_hbm, *rest):
    hps = rest[:n_tables]
    outs = rest[n_tables:n_tables + n_out]
    zb, src_all, dst_all, didx_g, rows = rest[n_tables + n_out:
                                              n_tables + n_out + 5]
    gsems = rest[n_tables + n_out + 5:n_tables + n_out + 5 + NB]
    ssems = rest[n_tables + n_out + 5 + NB:n_tables + n_out + 5 + 2 * NB]
    acc = rest[n_tables + n_out + 5 + 2 * NB]
    s = lax.axis_index("s")
    _zero_rows(zb, hrpt)
    base = s * EPT
    pltpu.sync_copy(src_hbm.at[pl.ds(base, EPT)], src_all)
    pltpu.sync_copy(dst_hbm.at[pl.ds(base, EPT)], dst_all)
    if not n_tables:

      def ones_body(i, _):
        for w in range(DC // 16):
          rows[0, i, pl.ds(w * 16, 16)] = jnp.ones((16,), jnp.float32)
        return 0
      lax.fori_loop(0, K, ones_body, 0)

    for h in range(n_ranges):
      lo = h * rw

      for c in range(n_out):
        pltpu.sync_copy(zb, acc.at[pl.ds(s * hrpt, hrpt)])
        plsc.subcore_barrier()

        def mask_batch(j, b):
          for w in range(K // 16):
            v = dst_all[pl.ds(j * K + w * 16, 16)]
            ok = jnp.logical_and(v >= lo, v < lo + rw)
            didx_g[b, pl.ds(w * 16, 16)] = jnp.where(ok, v - lo, rw)

        if n_tables:
          table = hps[c]

          def gather(j, b):
            pltpu.async_copy(
                table.at[src_all.at[pl.ds(j * K, K)]], rows.at[b], gsems[b])

          def gwait(j, b):
            pltpu.make_async_copy(
                table.at[src_all.at[pl.ds(j * K, K)]], rows.at[b],
                gsems[b]).wait()

          def scat(b):
            pltpu.async_copy(rows.at[b], acc.at[didx_g.at[b]], ssems[b],
                             add=True)

          def swait(b):
            pltpu.make_async_copy(rows.at[b], acc.at[didx_g.at[b]],
                                  ssems[b]).wait()

          for b in range(NB):
            gather(b, b)

          def grp(it, _):
            j0 = it * NB
            for b in range(NB):
              mask_batch(j0 + b, b)
            for b in range(NB):
              gwait(j0 + b, b)
              scat(b)
            for b in range(NB):
              swait(b)
              gather(j0 + NB + b, b)
            return 0
          lax.fori_loop(0, ngrp - 1, grp, 0)
          j0 = (ngrp - 1) * NB
          for b in range(NB):
            mask_batch(j0 + b, b)
          for b in range(NB):
            gwait(j0 + b, b)
            scat(b)
          for b in range(NB):
            swait(b)
        else:

          def grpd(it, _):
            j0 = it * NB
            for b in range(NB):
              mask_batch(j0 + b, b)
            for b in range(NB):
              pltpu.async_copy(rows.at[0], acc.at[didx_g.at[b]], ssems[b],
                               add=True)
            for b in range(NB):
              pltpu.make_async_copy(rows.at[0], acc.at[didx_g.at[b]],
                                    ssems[b]).wait()
            return 0
          lax.fori_loop(0, ngrp, grpd, 0)

        plsc.subcore_barrier()
        o = 0
        bi = 1 if not n_tables else 0
        while o < wrpt:
          ln = min(K, wrpt - o)
          stg = rows.at[bi].at[pl.ds(0, ln)]
          pltpu.sync_copy(acc.at[pl.ds(s * wrpt + o, ln)], stg)
          pltpu.sync_copy(stg, outs[c].at[pl.ds(lo + s * wrpt + o, ln)])
          o += ln
        plsc.subcore_barrier()

  return pl.kernel(
      body,
      out_type=tuple(
          jax.ShapeDtypeStruct((out_rows, DC), jnp.float32)
          for _ in range(n_out)),
      mesh=_MESH,
      scratch_types=[
          pltpu.VMEM((hrpt, DC), jnp.float32),
          pltpu.VMEM((EPT,), jnp.int32),
          pltpu.VMEM((EPT,), jnp.int32),
          pltpu.VMEM((NB, K), jnp.int32),
          pltpu.VMEM((NB, K, DC), jnp.float32),
      ] + [pltpu.SemaphoreType.DMA] * (2 * NB) + [
          pltpu.VMEM_SHARED((hd, DC), jnp.float32),
      ],
  )


_deg_sc = _make_sc(0, 4, 3200)
_agg_l1 = _make_sc(4, 3, 3456)
_agg_l2 = _make_sc(4, 3, 3456)
_agg_l3 = _make_sc(2, 3, 3456)


_R = 1000


def _dis(deg_ref):
  return lax.rsqrt(deg_ref[...][:, 0:1] + 1.0)


def _mm_first_body(x_ref, w_ref, deg_ref, out_ref):
  hp = jnp.dot(x_ref[...], w_ref[...],
               preferred_element_type=jnp.float32) * _dis(deg_ref)
  for cc in range(out_ref.shape[0]):
    out_ref[cc] = hp[:, cc * DC:(cc + 1) * DC]


def _mm_mid_body(s_ref, hp_ref, b_ref, w_ref, deg_ref, out_ref):
  dis = _dis(deg_ref)
  b = b_ref[...]
  parts = [
      jnp.maximum(dis * (s_ref[cc] + hp_ref[cc]) +
                  b[:, cc * DC:(cc + 1) * DC], 0.0)
      for cc in range(s_ref.shape[0])
  ]
  x = jnp.concatenate(parts, axis=1)
  hp = jnp.dot(x, w_ref[...], preferred_element_type=jnp.float32) * dis
  for cc in range(out_ref.shape[0]):
    out_ref[cc] = hp[:, cc * DC:(cc + 1) * DC]


def _final_body(s_ref, hp_ref, b_ref, deg_ref, out_ref):
  dis = _dis(deg_ref)
  b = b_ref[...]
  parts = [
      dis * (s_ref[cc] + hp_ref[cc]) + b[:, cc * DC:(cc + 1) * DC]
      for cc in range(s_ref.shape[0])
  ]
  out_ref[...] = jnp.concatenate(parts, axis=1)


def _chunk_spec(n_chunks):
  return pl.BlockSpec((n_chunks, _R, DC), lambda i: (0, i, 0))


def _full_spec(shape):
  ndim = len(shape)
  return pl.BlockSpec(shape, lambda i: (0,) * ndim)


_DEG_SPEC = pl.BlockSpec((_R, 16), lambda i: (i, 0))


def _mm_first(x, w, deg16, n_out):
  return pl.pallas_call(
      _mm_first_body,
      grid=(N // _R,),
      in_specs=[
          pl.BlockSpec((_R, x.shape[1]), lambda i: (i, 0)),
          _full_spec(w.shape),
          _DEG_SPEC,
      ],
      out_specs=_chunk_spec(n_out),
      out_shape=jax.ShapeDtypeStruct((n_out, N, DC), jnp.float32),
  )(x, w, deg16)


def _mm_mid(s_chunks, hp, b, w, deg16, n_out):
  n_in = hp.shape[0]
  return pl.pallas_call(
      _mm_mid_body,
      grid=(N // _R,),
      in_specs=[
          _chunk_spec(n_in),
          _chunk_spec(n_in),
          _full_spec((1, b.shape[1])),
          _full_spec(w.shape),
          _DEG_SPEC,
      ],
      out_specs=_chunk_spec(n_out),
      out_shape=jax.ShapeDtypeStruct((n_out, N, DC), jnp.float32),
  )(s_chunks, hp, b, w, deg16)


def _final(s_chunks, hp, b, deg16):
  n_in = hp.shape[0]
  return pl.pallas_call(
      _final_body,
      grid=(N // _R,),
      in_specs=[
          _chunk_spec(n_in),
          _chunk_spec(n_in),
          _full_spec((1, b.shape[1])),
          _DEG_SPEC,
      ],
      out_specs=pl.BlockSpec((_R, n_in * DC), lambda i: (i, 0)),
      out_shape=jax.ShapeDtypeStruct((N, n_in * DC), jnp.float32),
  )(s_chunks, hp, b, deg16)


def kernel(edge_indices, features, W1, b1, W2, b2, W3, b3):
  src = edge_indices[0].astype(jnp.int32)
  dst = edge_indices[1].astype(jnp.int32)
  b1r = b1.reshape(1, -1)
  b2r = b2.reshape(1, -1)
  b3r = b3.reshape(1, -1)

  (degfull,) = _deg_sc(src, dst)
  deg16 = degfull[:N, :16]

  def agg(fn, hp):
    outs = fn(src, dst, *[hp[i] for i in range(hp.shape[0])])
    return jnp.stack([o[:N] for o in outs])

  hp1 = _mm_first(features, W1, deg16, 4)
  s1 = agg(_agg_l1, hp1)
  hp2 = _mm_mid(s1, hp1, b1r, W2, deg16, 4)
  s2 = agg(_agg_l2, hp2)
  hp3 = _mm_mid(s2, hp2, b2r, W3, deg16, 2)
  s3 = agg(_agg_l3, hp3)
  return _final(s3, hp3, b3r, deg16)


# speedup vs baseline: 1.0059x; 1.0059x over previous
import functools

import jax
import jax.numpy as jnp
from jax import lax
from jax.experimental import pallas as pl
from jax.experimental.pallas import tpu as pltpu
from jax.experimental.pallas import tpu_sc as plsc

N = 10000
NP = 10240
E = 160000
NS = 16
K = 80
EPT = E // NS
DC = 128

_MESH = plsc.VectorSubcoreMesh(
    core_axis_name="c", subcore_axis_name="s", num_cores=1, num_subcores=NS)


def _zero_rows(buf, rows):
  def body(i, _):
    for w in range(DC // 16):
      buf[i, pl.ds(w * 16, 16)] = jnp.zeros((16,), jnp.float32)
    return 0
  lax.fori_loop(0, rows, body, 0)


NB = 5
_NSTEPS = EPT // K


def _make_sc(n_tables, n_ranges, rw):
  hd = rw + 128
  hrpt = hd // NS
  wrpt = rw // NS
  assert hd % 128 == 0 and wrpt % 8 == 0 and n_ranges * rw >= NP
  out_rows = n_ranges * rw
  n_out = max(n_tables, 1)
  ngrp = _NSTEPS // NB

  def body(src_hbm, dst_hbm, *rest):
    hps = rest[:n_tables]
    outs = rest[n_tables:n_tables + n_out]
    zb, src_all, dst_all, didx_g, rows = rest[n_tables + n_out:
                                              n_tables + n_out + 5]
    gsems = rest[n_tables + n_out + 5:n_tables + n_out + 5 + NB]
    acc = rest[n_tables + n_out + 5 + NB]
    s = lax.axis_index("s")
    _zero_rows(zb, hrpt)
    base = s * EPT
    pltpu.sync_copy(src_hbm.at[pl.ds(base, EPT)], src_all)
    pltpu.sync_copy(dst_hbm.at[pl.ds(base, EPT)], dst_all)
    if not n_tables:

      def ones_body(i, _):
        for w in range(DC // 16):
          rows[0, i, pl.ds(w * 16, 16)] = jnp.ones((16,), jnp.float32)
        return 0
      lax.fori_loop(0, K, ones_body, 0)

    for h in range(n_ranges):
      lo = h * rw

      for c in range(n_out):
        pltpu.sync_copy(zb, acc.at[pl.ds(s * hrpt, hrpt)])
        plsc.subcore_barrier()

        def mask_batch(j, b):
          for w in range(K // 16):
            v = dst_all[pl.ds(j * K + w * 16, 16)]
            ok = jnp.logical_and(v >= lo, v < lo + rw)
            didx_g[b, pl.ds(w * 16, 16)] = jnp.where(ok, v - lo, rw)

        if n_tables:
          table = hps[c]

          def gather(j, b):
            pltpu.async_copy(
                table.at[src_all.at[pl.ds(j * K, K)]], rows.at[b], gsems[b])

          def gwait(j, b):
            pltpu.make_async_copy(
                table.at[src_all.at[pl.ds(j * K, K)]], rows.at[b],
                gsems[b]).wait()

          for b in range(NB):
            gather(b, b)

          def grp(it, _):
            j0 = it * NB
            for b in range(NB):
              j = j0 + b
              mask_batch(j, b)
              gwait(j, b)
              pltpu.sync_copy(rows.at[b], acc.at[didx_g.at[b]], add=True)
              gather(j + NB, b)
            return 0
          lax.fori_loop(0, ngrp - 1, grp, 0)
          j0 = (ngrp - 1) * NB
          for b in range(NB):
            mask_batch(j0 + b, b)
            gwait(j0 + b, b)
            pltpu.sync_copy(rows.at[b], acc.at[didx_g.at[b]], add=True)
        else:

          def grpd(it, _):
            j0 = it * NB
            for b in range(NB):
              mask_batch(j0 + b, b)
              pltpu.sync_copy(rows.at[0], acc.at[didx_g.at[b]], add=True)
            return 0
          lax.fori_loop(0, ngrp, grpd, 0)

        plsc.subcore_barrier()
        o = 0
        bi = 1 if not n_tables else 0
        while o < wrpt:
          ln = min(K, wrpt - o)
          stg = rows.at[bi].at[pl.ds(0, ln)]
          pltpu.sync_copy(acc.at[pl.ds(s * wrpt + o, ln)], stg)
          pltpu.sync_copy(stg, outs[c].at[pl.ds(lo + s * wrpt + o, ln)])
          o += ln
        plsc.subcore_barrier()

  return pl.kernel(
      body,
      out_type=tuple(
          jax.ShapeDtypeStruct((out_rows, DC), jnp.float32)
          for _ in range(n_out)),
      mesh=_MESH,
      scratch_types=[
          pltpu.VMEM((hrpt, DC), jnp.float32),
          pltpu.VMEM((EPT,), jnp.int32),
          pltpu.VMEM((EPT,), jnp.int32),
          pltpu.VMEM((NB, K), jnp.int32),
          pltpu.VMEM((NB, K, DC), jnp.float32),
      ] + [pltpu.SemaphoreType.DMA] * NB + [
          pltpu.VMEM_SHARED((hd, DC), jnp.float32),
      ],
  )


_deg_sc = _make_sc(0, 4, 3200)
_agg_l1 = _make_sc(4, 3, 3456)
_agg_l2 = _make_sc(4, 3, 3456)
_agg_l3 = _make_sc(2, 3, 3456)


_R = 1000


def _dis(deg_ref):
  return lax.rsqrt(deg_ref[...][:, 0:1] + 1.0)


def _mm_first_body(x_ref, w_ref, deg_ref, out_ref):
  hp = jnp.dot(x_ref[...], w_ref[...],
               preferred_element_type=jnp.float32) * _dis(deg_ref)
  for cc in range(out_ref.shape[0]):
    out_ref[cc] = hp[:, cc * DC:(cc + 1) * DC]


def _mm_mid_body(s_ref, hp_ref, b_ref, w_ref, deg_ref, out_ref):
  dis = _dis(deg_ref)
  b = b_ref[...]
  parts = [
      jnp.maximum(dis * (s_ref[cc] + hp_ref[cc]) +
                  b[:, cc * DC:(cc + 1) * DC], 0.0)
      for cc in range(s_ref.shape[0])
  ]
  x = jnp.concatenate(parts, axis=1)
  hp = jnp.dot(x, w_ref[...], preferred_element_type=jnp.float32) * dis
  for cc in range(out_ref.shape[0]):
    out_ref[cc] = hp[:, cc * DC:(cc + 1) * DC]


def _final_body(s_ref, hp_ref, b_ref, deg_ref, out_ref):
  dis = _dis(deg_ref)
  b = b_ref[...]
  parts = [
      dis * (s_ref[cc] + hp_ref[cc]) + b[:, cc * DC:(cc + 1) * DC]
      for cc in range(s_ref.shape[0])
  ]
  out_ref[...] = jnp.concatenate(parts, axis=1)


def _chunk_spec(n_chunks):
  return pl.BlockSpec((n_chunks, _R, DC), lambda i: (0, i, 0))


def _full_spec(shape):
  ndim = len(shape)
  return pl.BlockSpec(shape, lambda i: (0,) * ndim)


_DEG_SPEC = pl.BlockSpec((_R, 16), lambda i: (i, 0))


def _mm_first(x, w, deg16, n_out):
  return pl.pallas_call(
      _mm_first_body,
      grid=(N // _R,),
      in_specs=[
          pl.BlockSpec((_R, x.shape[1]), lambda i: (i, 0)),
          _full_spec(w.shape),
          _DEG_SPEC,
      ],
      out_specs=_chunk_spec(n_out),
      out_shape=jax.ShapeDtypeStruct((n_out, N, DC), jnp.float32),
  )(x, w, deg16)


def _mm_mid(s_chunks, hp, b, w, deg16, n_out):
  n_in = hp.shape[0]
  return pl.pallas_call(
      _mm_mid_body,
      grid=(N // _R,),
      in_specs=[
          _chunk_spec(n_in),
          _chunk_spec(n_in),
          _full_spec((1, b.shape[1])),
          _full_spec(w.shape),
          _DEG_SPEC,
      ],
      out_specs=_chunk_spec(n_out),
      out_shape=jax.ShapeDtypeStruct((n_out, N, DC), jnp.float32),
  )(s_chunks, hp, b, w, deg16)


def _final(s_chunks, hp, b, deg16):
  n_in = hp.shape[0]
  return pl.pallas_call(
      _final_body,
      grid=(N // _R,),
      in_specs=[
          _chunk_spec(n_in),
          _chunk_spec(n_in),
          _full_spec((1, b.shape[1])),
          _DEG_SPEC,
      ],
      out_specs=pl.BlockSpec((_R, n_in * DC), lambda i: (i, 0)),
      out_shape=jax.ShapeDtypeStruct((N, n_in * DC), jnp.float32),
  )(s_chunks, hp, b, deg16)


def kernel(edge_indices, features, W1, b1, W2, b2, W3, b3):
  src = edge_indices[0].astype(jnp.int32)
  dst = edge_indices[1].astype(jnp.int32)
  b1r = b1.reshape(1, -1)
  b2r = b2.reshape(1, -1)
  b3r = b3.reshape(1, -1)

  (degfull,) = _deg_sc(src, dst)
  deg16 = degfull[:N, :16]

  def agg(fn, hp):
    outs = fn(src, dst, *[hp[i] for i in range(hp.shape[0])])
    return jnp.stack([o[:N] for o in outs])

  hp1 = _mm_first(features, W1, deg16, 4)
  s1 = agg(_agg_l1, hp1)
  hp2 = _mm_mid(s1, hp1, b1r, W2, deg16, 4)
  s2 = agg(_agg_l2, hp2)
  hp3 = _mm_mid(s2, hp2, b2r, W3, deg16, 2)
  s3 = agg(_agg_l3, hp3)
  return _final(s3, hp3, b3r, deg16)
